# baseline (speedup 1.0000x reference)
import jax
import jax.numpy as jnp
from jax.experimental import pallas as pl

N = 10000
K = 32
_BLK = 10000


def _contract(x, w):
    return jax.lax.dot_general(
        x, w, (((1,), (1,)), ((), ())), preferred_element_type=jnp.float32
    )


def _mlp_kernel(a_ref, W1_ref, b1_ref, We1_ref, be1_ref, We2_ref, be2_ref,
                Wd1_ref, bd1_ref, Wd2_ref, bd2_ref, Wo_ref, bo_ref, o_ref):
    a = a_ref[...]
    h = jnp.tanh(_contract(a, W1_ref[...]) + b1_ref[...])
    h = jnp.tanh(_contract(h, We1_ref[...]) + be1_ref[...])
    e = jnp.tanh(_contract(h, We2_ref[...]) + be2_ref[...])
    d = jnp.tanh(_contract(e, Wd1_ref[...]) + bd1_ref[...])
    d = _contract(d, Wd2_ref[...]) + bd2_ref[...]
    o_ref[...] = jax.nn.sigmoid(_contract(jnp.tanh(d), Wo_ref[...]) + bo_ref[...])


def _pad_rows(w, rows):
    return jnp.zeros((rows, w.shape[1]), w.dtype).at[: w.shape[0]].set(w)


def _pad_bias(b, n):
    return jnp.zeros((1, n), b.dtype).at[0, : b.shape[0]].set(b)


def kernel(x, edge_index, edge_attr, W_ih, W_hh, b_ih, b_hh, W1, b1, Wg, bg,
           We1, be1, We2, be2, Wd1, bd1, Wd2, bd2, Wo, bo):
    a = edge_attr.reshape(N, K)

    We2p = _pad_rows(We2, 8)
    be2p = _pad_bias(be2, 8)
    Wd1p = jnp.zeros((128, 8), Wd1.dtype).at[:, :6].set(Wd1)
    Wop = _pad_rows(Wo, 8)
    bop = _pad_bias(bo, 8)

    def _full(arr):
        return pl.BlockSpec(arr.shape, lambda i: (0,) * arr.ndim)

    weights = (W1, b1.reshape(1, -1), We1, be1.reshape(1, -1),
               We2p, be2p, Wd1p, bd1.reshape(1, -1),
               Wd2, bd2.reshape(1, -1), Wop, bop)

    out = pl.pallas_call(
        _mlp_kernel,
        grid=(N // _BLK,),
        in_specs=[pl.BlockSpec((_BLK, K), lambda i: (i, 0))]
                 + [_full(w) for w in weights],
        out_specs=pl.BlockSpec((_BLK, 8), lambda i: (i, 0)),
        out_shape=jax.ShapeDtypeStruct((N, 8), jnp.float32),
    )(a, *weights)
    return out[:, 0]


# speedup vs baseline: 1.2016x; 1.2016x over previous
import jax
import jax.numpy as jnp
from jax.experimental import pallas as pl
from jax.experimental.pallas import tpu as pltpu

N = 10000
K = 32


def _contract(x, w):
    return jax.lax.dot_general(
        x, w, (((1,), (1,)), ((), ())), preferred_element_type=jnp.float32
    )


def _mlp_kernel(a_ref, W1_ref, b1_ref, We1_ref, be1_ref, We2_ref, be2_ref,
                Wd1_ref, bd1_ref, Wd2_ref, bd2_ref, Wo_ref, bo_ref, o_ref):
    f32 = jnp.float32
    We2p = jnp.concatenate([We2_ref[...], jnp.zeros((2, 128), f32)], axis=0)
    be2p = jnp.concatenate([be2_ref[...], jnp.zeros((1, 2), f32)], axis=1)
    Wd1p = jnp.concatenate([Wd1_ref[...], jnp.zeros((128, 2), f32)], axis=1)
    Wop = jnp.concatenate([Wo_ref[...], jnp.zeros((7, 256), f32)], axis=0)

    a = a_ref[...]
    h = jnp.tanh(_contract(a, W1_ref[...]) + b1_ref[...])
    h = jnp.tanh(_contract(h, We1_ref[...]) + be1_ref[...])
    e = jnp.tanh(_contract(h, We2p) + be2p)
    d = jnp.tanh(_contract(e, Wd1p) + bd1_ref[...])
    d = _contract(d, Wd2_ref[...]) + bd2_ref[...]
    z = _contract(jnp.tanh(d), Wop) + bo_ref[0]
    o_ref[...] = jax.nn.sigmoid(z[:, :1])


def kernel(x, edge_index, edge_attr, W_ih, W_hh, b_ih, b_hh, W1, b1, Wg, bg,
           We1, be1, We2, be2, Wd1, bd1, Wd2, bd2, Wo, bo):
    a = edge_attr.reshape(N, K)

    def _vmem(arr):
        return pl.BlockSpec(arr.shape, lambda *_: (0,) * arr.ndim)

    weights = (W1, b1.reshape(1, -1), We1, be1.reshape(1, -1),
               We2, be2.reshape(1, -1), Wd1, bd1.reshape(1, -1),
               Wd2, bd2.reshape(1, -1), Wo)

    out = pl.pallas_call(
        _mlp_kernel,
        grid=(1,),
        in_specs=[_vmem(a)] + [_vmem(w) for w in weights]
                 + [pl.BlockSpec(memory_space=pltpu.SMEM)],
        out_specs=pl.BlockSpec((N, 1), lambda *_: (0, 0)),
        out_shape=jax.ShapeDtypeStruct((N, 1), jnp.float32),
    )(a, *weights, bo)
    return out[:, 0]


# speedup vs baseline: 1.2042x; 1.0021x over previous
import jax
import jax.numpy as jnp
from jax.experimental import pallas as pl
from jax.experimental.pallas import tpu as pltpu

N = 10000
K = 32


def _contract(x, w):
    return jax.lax.dot_general(
        x, w, (((1,), (1,)), ((), ())), preferred_element_type=jnp.float32
    )


def _mlp_kernel(a_ref, W1_ref, b1_ref, We1_ref, be1_ref, We2_ref, be2_ref,
                Wd1_ref, bd1_ref, Wd2_ref, bd2_ref, Wo_ref, bo_ref, o_ref):
    f32 = jnp.float32
    We2p = jnp.concatenate([We2_ref[...], jnp.zeros((2, 128), f32)], axis=0)
    be2p = jnp.concatenate([be2_ref[...], jnp.zeros((2,), f32)]).reshape(1, 8)
    Wd1p = jnp.concatenate([Wd1_ref[...], jnp.zeros((128, 2), f32)], axis=1)
    Wop = jnp.concatenate([Wo_ref[...], jnp.zeros((7, 256), f32)], axis=0)

    a = a_ref[...]
    h = jnp.tanh(_contract(a, W1_ref[...]) + b1_ref[...].reshape(1, -1))
    h = jnp.tanh(_contract(h, We1_ref[...]) + be1_ref[...].reshape(1, -1))
    e = jnp.tanh(_contract(h, We2p) + be2p)
    d = jnp.tanh(_contract(e, Wd1p) + bd1_ref[...].reshape(1, -1))
    d = _contract(d, Wd2_ref[...]) + bd2_ref[...].reshape(1, -1)
    z = _contract(jnp.tanh(d), Wop) + bo_ref[0]
    o_ref[...] = jax.nn.sigmoid(z[:, :1])


def kernel(x, edge_index, edge_attr, W_ih, W_hh, b_ih, b_hh, W1, b1, Wg, bg,
           We1, be1, We2, be2, Wd1, bd1, Wd2, bd2, Wo, bo):
    a = edge_attr.reshape(N, K)

    def _vmem(arr):
        return pl.BlockSpec(arr.shape, lambda *_: (0,) * arr.ndim)

    weights = (W1, b1, We1, be1, We2, be2, Wd1, bd1, Wd2, bd2, Wo)

    out = pl.pallas_call(
        _mlp_kernel,
        grid=(1,),
        in_specs=[_vmem(a)] + [_vmem(w) for w in weights]
                 + [pl.BlockSpec(memory_space=pltpu.SMEM)],
        out_specs=pl.BlockSpec((N, 1), lambda *_: (0, 0)),
        out_shape=jax.ShapeDtypeStruct((N, 1), jnp.float32),
    )(a, *weights, bo)
    return out[:, 0]


# speedup vs baseline: 1.2426x; 1.0319x over previous
import jax
import jax.numpy as jnp
from jax.experimental import pallas as pl
from jax.experimental.pallas import tpu as pltpu

N = 10000
K = 32


def _contract(x, w):
    return jax.lax.dot_general(
        x, w, (((1,), (1,)), ((), ())), preferred_element_type=jnp.float32
    )


def _mlp_kernel(a_ref, W1_ref, b1_ref, We1_ref, be1_ref, We2_ref, be2_ref,
                Wd1_ref, bd1_ref, Wd2_ref, bd2_ref, Wo_ref, bo_ref, o_ref):
    f32 = jnp.float32
    We2p = jnp.concatenate([We2_ref[...], jnp.zeros((2, 128), f32)], axis=0)
    be2p = jnp.concatenate([be2_ref[...], jnp.zeros((2,), f32)]).reshape(1, 8)
    Wd1p = jnp.concatenate([Wd1_ref[...], jnp.zeros((128, 2), f32)], axis=1)
    Wop = jnp.concatenate([Wo_ref[...], jnp.zeros((7, 256), f32)], axis=0)

    a = a_ref[...]
    h = jnp.tanh(_contract(a, W1_ref[...]) + b1_ref[...].reshape(1, -1))
    h = jnp.tanh(_contract(h, We1_ref[...]) + be1_ref[...].reshape(1, -1))
    e = jnp.tanh(_contract(h, We2p) + be2p)
    d = jnp.tanh(_contract(e, Wd1p) + bd1_ref[...].reshape(1, -1))
    d = _contract(d, Wd2_ref[...]) + bd2_ref[...].reshape(1, -1)
    z = _contract(jnp.tanh(d), Wop) + bo_ref[0]
    o_ref[...] = jax.nn.sigmoid(z[:, :1])


def kernel(x, edge_index, edge_attr, W_ih, W_hh, b_ih, b_hh, W1, b1, Wg, bg,
           We1, be1, We2, be2, Wd1, bd1, Wd2, bd2, Wo, bo):
    a = edge_attr.reshape(N, K)

    def _vmem(arr):
        return pl.BlockSpec(arr.shape, lambda *_: (0,) * arr.ndim)

    weights = (W1, b1, We1, be1, We2, be2, Wd1, bd1, Wd2, bd2, Wo)

    blk = 2000
    out = pl.pallas_call(
        _mlp_kernel,
        grid=(N // blk,),
        in_specs=[pl.BlockSpec((blk, K), lambda i: (i, 0))]
                 + [_vmem(w) for w in weights]
                 + [pl.BlockSpec(memory_space=pltpu.SMEM)],
        out_specs=pl.BlockSpec((blk, 1), lambda i: (i, 0)),
        out_shape=jax.ShapeDtypeStruct((N, 1), jnp.float32),
    )(a, *weights, bo)
    return out[:, 0]


# speedup vs baseline: 1.2452x; 1.0021x over previous
import jax
import jax.numpy as jnp
from jax.experimental import pallas as pl
from jax.experimental.pallas import tpu as pltpu

N = 10000
K = 32
NP = N // 4


def _contract(x, w):
    return jax.lax.dot_general(
        x, w, (((1,), (1,)), ((), ())), preferred_element_type=jnp.float32
    )


def _mlp_kernel(xp_ref, W1_ref, b1_ref, We1_ref, be1_ref, We2_ref, be2_ref,
                Wd1_ref, bd1_ref, Wd2_ref, bd2_ref, Wo_ref, bo_ref, o_ref,
                w1big_ref):
    f32 = jnp.float32
    w1big_ref[...] = jnp.zeros((128, 1024), f32)
    w1t = W1_ref[...].T
    for r in range(4):
        w1big_ref[32 * r:32 * (r + 1), 256 * r:256 * (r + 1)] = w1t

    b1 = b1_ref[...].reshape(1, -1)
    b1big = jnp.concatenate([b1, b1, b1, b1], axis=1)

    hp = jnp.tanh(jnp.dot(xp_ref[...], w1big_ref[...],
                          preferred_element_type=f32) + b1big)
    h = hp.reshape(N, 256)

    We2p = jnp.concatenate([We2_ref[...], jnp.zeros((2, 128), f32)], axis=0)
    be2p = jnp.concatenate([be2_ref[...], jnp.zeros((2,), f32)]).reshape(1, 8)
    Wd1p = jnp.concatenate([Wd1_ref[...], jnp.zeros((128, 2), f32)], axis=1)
    Wop = jnp.concatenate([Wo_ref[...], jnp.zeros((7, 256), f32)], axis=0)

    h = jnp.tanh(_contract(h, We1_ref[...]) + be1_ref[...].reshape(1, -1))
    e = jnp.tanh(_contract(h, We2p) + be2p)
    d = jnp.tanh(_contract(e, Wd1p) + bd1_ref[...].reshape(1, -1))
    d = _contract(d, Wd2_ref[...]) + bd2_ref[...].reshape(1, -1)
    z = _contract(jnp.tanh(d), Wop) + bo_ref[0]
    o_ref[...] = jax.nn.sigmoid(z[:, :1])


def kernel(x, edge_index, edge_attr, W_ih, W_hh, b_ih, b_hh, W1, b1, Wg, bg,
           We1, be1, We2, be2, Wd1, bd1, Wd2, bd2, Wo, bo):
    xp = edge_attr.reshape(NP, 128)

    def _vmem(arr):
        return pl.BlockSpec(arr.shape, lambda *_: (0,) * arr.ndim)

    weights = (W1, b1, We1, be1, We2, be2, Wd1, bd1, Wd2, bd2, Wo)

    out = pl.pallas_call(
        _mlp_kernel,
        grid=(1,),
        in_specs=[_vmem(xp)] + [_vmem(w) for w in weights]
                 + [pl.BlockSpec(memory_space=pltpu.SMEM)],
        out_specs=pl.BlockSpec((N, 1), lambda *_: (0, 0)),
        out_shape=jax.ShapeDtypeStruct((N, 1), jnp.float32),
        scratch_shapes=[pltpu.VMEM((128, 1024), jnp.float32)],
    )(xp, *weights, bo)
    return out[:, 0]


# speedup vs baseline: 1.4249x; 1.1443x over previous
import jax
import jax.numpy as jnp
from jax.experimental import pallas as pl
from jax.experimental.pallas import tpu as pltpu

N = 10000
K = 32
NP = N // 4


def _contract(x, w):
    return jax.lax.dot_general(
        x, w, (((1,), (1,)), ((), ())), preferred_element_type=jnp.float32
    )


def _mlp_kernel(xp_ref, W1_ref, b1_ref, We1_ref, be1_ref, We2_ref, be2_ref,
                Wd1_ref, bd1_ref, Wd2_ref, bd2_ref, Wo_ref, bo_ref, o_ref,
                w1big_ref):
    f32 = jnp.float32
    w1big_ref[...] = jnp.zeros((128, 1024), f32)
    w1t = W1_ref[...].T
    for r in range(4):
        w1big_ref[32 * r:32 * (r + 1), 256 * r:256 * (r + 1)] = w1t

    b1 = b1_ref[...].reshape(1, -1)
    b1big = jnp.concatenate([b1, b1, b1, b1], axis=1)

    hp = jnp.tanh(jnp.dot(xp_ref[...], w1big_ref[...],
                          preferred_element_type=f32) + b1big)
    h = hp.reshape(N, 256)

    We2p = jnp.concatenate([We2_ref[...], jnp.zeros((2, 128), f32)], axis=0)
    be2p = jnp.concatenate([be2_ref[...], jnp.zeros((2,), f32)]).reshape(1, 8)
    Wd1p = jnp.concatenate([Wd1_ref[...], jnp.zeros((128, 2), f32)], axis=1)
    Wop = jnp.concatenate([Wo_ref[...], jnp.zeros((7, 256), f32)], axis=0)

    h = jnp.tanh(_contract(h, We1_ref[...]) + be1_ref[...].reshape(1, -1))
    e = jnp.tanh(_contract(h, We2p) + be2p)
    d = jnp.tanh(_contract(e, Wd1p) + bd1_ref[...].reshape(1, -1))
    d = _contract(d, Wd2_ref[...]) + bd2_ref[...].reshape(1, -1)
    z = _contract(jnp.tanh(d), Wop) + bo_ref[0]
    o_ref[...] = jax.nn.sigmoid(z[:, 0])


def kernel(x, edge_index, edge_attr, W_ih, W_hh, b_ih, b_hh, W1, b1, Wg, bg,
           We1, be1, We2, be2, Wd1, bd1, Wd2, bd2, Wo, bo):
    xp = edge_attr.reshape(NP, 128)

    def _vmem(arr):
        return pl.BlockSpec(arr.shape, lambda *_: (0,) * arr.ndim)

    weights = (W1, b1, We1, be1, We2, be2, Wd1, bd1, Wd2, bd2, Wo)

    out = pl.pallas_call(
        _mlp_kernel,
        grid=(1,),
        in_specs=[_vmem(xp)] + [_vmem(w) for w in weights]
                 + [pl.BlockSpec(memory_space=pltpu.SMEM)],
        out_specs=pl.BlockSpec((N,), lambda *_: (0,)),
        out_shape=jax.ShapeDtypeStruct((N,), jnp.float32),
        scratch_shapes=[pltpu.VMEM((128, 1024), jnp.float32)],
    )(xp, *weights, bo)
    return out


# speedup vs baseline: 1.5009x; 1.0533x over previous
import jax
import jax.numpy as jnp
from jax.experimental import pallas as pl
from jax.experimental.pallas import tpu as pltpu

N = 10000
K = 32
NP = N // 4
NPP = 2560


def _contract(x, w):
    return jax.lax.dot_general(
        x, w, (((1,), (1,)), ((), ())), preferred_element_type=jnp.float32
    )


def _mlp_kernel(xp_ref, W1_ref, b1_ref, We1_ref, be1_ref, We2_ref, be2_ref,
                Wd1_ref, bd1_ref, Wd2_ref, bd2_ref, Wo_ref, bo_ref, o_ref,
                w1big_ref):
    f32 = jnp.float32
    w1big_ref[...] = jnp.zeros((128, 1024), f32)
    w1t = W1_ref[...].T
    for r in range(4):
        w1big_ref[32 * r:32 * (r + 1), 256 * r:256 * (r + 1)] = w1t

    b1 = b1_ref[...].reshape(1, -1)
    b1big = jnp.concatenate([b1, b1, b1, b1], axis=1)

    hp = jnp.tanh(jnp.dot(xp_ref[...], w1big_ref[...],
                          preferred_element_type=f32) + b1big)
    h = hp.reshape(4 * NPP, 256)

    We2p = jnp.concatenate([We2_ref[...], jnp.zeros((2, 128), f32)], axis=0)
    be2p = jnp.concatenate([be2_ref[...], jnp.zeros((2,), f32)]).reshape(1, 8)
    Wd1p = jnp.concatenate([Wd1_ref[...], jnp.zeros((128, 2), f32)], axis=1)
    Wop = jnp.concatenate([Wo_ref[...], jnp.zeros((7, 256), f32)], axis=0)

    h = jnp.tanh(_contract(h, We1_ref[...]) + be1_ref[...].reshape(1, -1))
    e = jnp.tanh(_contract(h, We2p) + be2p)
    d = jnp.tanh(_contract(e, Wd1p) + bd1_ref[...].reshape(1, -1))
    d = _contract(d, Wd2_ref[...]) + bd2_ref[...].reshape(1, -1)
    z = _contract(jnp.tanh(d), Wop) + bo_ref[0]
    o_ref[...] = jax.nn.sigmoid(z[:N, 0])


def kernel(x, edge_index, edge_attr, W_ih, W_hh, b_ih, b_hh, W1, b1, Wg, bg,
           We1, be1, We2, be2, Wd1, bd1, Wd2, bd2, Wo, bo):
    xp = jnp.concatenate(
        [edge_attr.reshape(NP, 128), jnp.zeros((NPP - NP, 128), jnp.float32)],
        axis=0)

    def _vmem(arr):
        return pl.BlockSpec(arr.shape, lambda *_: (0,) * arr.ndim)

    weights = (W1, b1, We1, be1, We2, be2, Wd1, bd1, Wd2, bd2, Wo)

    out = pl.pallas_call(
        _mlp_kernel,
        grid=(1,),
        in_specs=[_vmem(xp)] + [_vmem(w) for w in weights]
                 + [pl.BlockSpec(memory_space=pltpu.SMEM)],
        out_specs=pl.BlockSpec((N,), lambda *_: (0,)),
        out_shape=jax.ShapeDtypeStruct((N,), jnp.float32),
        scratch_shapes=[pltpu.VMEM((128, 1024), jnp.float32)],
    )(xp, *weights, bo)
    return out
